# baseline (speedup 1.0000x reference)
import functools

import jax
import jax.numpy as jnp
from jax.experimental import pallas as pl
from jax.experimental.pallas import tpu as pltpu

_C0 = 0.28209479177387814
_C1 = 0.48860251190291987
_C2a = 1.0925484305920792
_C2b = 0.94617469575755997
_C2c = 0.31539156525251999
_C2d = 0.54627421529603959
_C3a = 0.59004358992664352
_C3b = 2.8906114426405538
_C3c = 0.45704579946446572
_C3d = 0.3731763325901154
_C3e = 1.4453057213202769


def _sh_cols(d):
    x, y, z = d[:, 0:1], d[:, 1:2], d[:, 2:3]
    xx, yy, zz = x * x, y * y, z * z
    xy, yz, xz = x * y, y * z, x * z
    return [
        _C0 * jnp.ones_like(x),
        -_C1 * y,
        _C1 * z,
        -_C1 * x,
        _C2a * xy,
        -_C2a * yz,
        _C2b * zz - _C2c,
        -_C2a * xz,
        _C2d * (xx - yy),
        -_C3a * y * (3.0 * xx - yy),
        _C3b * xy * z,
        -_C3c * y * (4.0 * zz - xx - yy),
        _C3d * z * (2.0 * zz - 3.0 * xx - 3.0 * yy),
        -_C3c * x * (4.0 * zz - xx - yy),
        _C3e * z * (xx - yy),
        -_C3a * x * (xx - 3.0 * yy),
    ]


def _body(cf_ref, mask_ref, rays_ref, out_ref):
    d = rays_ref[...]
    inv = jax.lax.rsqrt(jnp.sum(d * d, axis=1, keepdims=True) + 1e-24)
    cols = _sh_cols(d * inv)
    sh = jnp.concatenate(cols, axis=1)
    cf = cf_ref[...]
    m = mask_ref[...].astype(jnp.float32)
    outs = []
    for c in range(3):
        prod = cf[:, :, 1 + 16 * c:17 + 16 * c] * sh[:, None, :]
        outs.append(jnp.sum(prod, axis=-1, keepdims=True))
    out_ref[...] = jnp.concatenate(outs, axis=-1) * m[:, :, None]


@functools.partial(jax.jit, static_argnames=("interpret",))
def kernel(color_features, mask, rays_d, interpret=False):
    B, N = mask.shape
    W = color_features.shape[-1]
    cf = color_features.reshape(B, N, W)
    Bb = 256
    out = pl.pallas_call(
        _body,
        grid=(B // Bb,),
        in_specs=[
            pl.BlockSpec((Bb, N, W), lambda i: (i, 0, 0)),
            pl.BlockSpec((Bb, N), lambda i: (i, 0)),
            pl.BlockSpec((Bb, 3), lambda i: (i, 0)),
        ],
        out_specs=pl.BlockSpec((Bb, N, 3), lambda i: (i, 0, 0)),
        out_shape=jax.ShapeDtypeStruct((B, N, 3), jnp.float32),
        interpret=interpret,
    )(cf, mask, rays_d)
    return out
